# baseline (speedup 1.0000x reference)
import jax
import jax.numpy as jnp
from jax import lax
from jax.experimental import pallas as pl
from jax.experimental.pallas import tpu as pltpu

_BETA = 0.25
_EPS = 1e-10


def _vq_body(e_ref, ze_ref, zd_ref, e2_ref, zq_ref, idx_ref, loss_ref, pp_ref,
             counts_ref, lacc_ref, ebf_ref, mm_ref):
    i = pl.program_id(0)
    nsteps = pl.num_programs(0)

    e2 = e2_ref[...]
    kcodes = e2.shape[0]
    cdim, tb = ze_ref.shape[1], ze_ref.shape[2]
    ntok = nsteps * tb

    @pl.when(i == 0)
    def _init():
        counts_ref[...] = jnp.zeros_like(counts_ref)
        lacc_ref[0, 0] = 0.0
        ebf_ref[...] = e_ref[...].astype(jnp.bfloat16)
        mm_ref[0] = lax.dot_general(
            ebf_ref[...], ze_ref[0].astype(jnp.bfloat16),
            (((1,), (0,)), ((), ())), preferred_element_type=jnp.float32)

    zblk = ze_ref[0]
    mm = mm_ref[i % 2]
    z2k = jnp.sum(zblk * zblk, axis=0, keepdims=True)
    dist = (z2k + e2) - 2.0 * mm
    minv = jnp.min(dist, axis=0, keepdims=True)
    iota = lax.broadcasted_iota(jnp.int32, (kcodes, tb), 0).astype(jnp.float32)
    idxf = jnp.min(jnp.where(dist == minv, iota, float(kcodes)),
                   axis=0, keepdims=True)
    idx_ref[0] = idxf.astype(jnp.int32)

    onehot = (iota == idxf).astype(jnp.bfloat16)
    ones = jnp.ones((tb, 1), jnp.bfloat16)
    counts_ref[...] += lax.dot_general(
        onehot, ones, (((1,), (0,)), ((), ())),
        preferred_element_type=jnp.float32)
    zq = lax.dot_general(ebf_ref[...], onehot, (((0,), (0,)), ((), ())),
                         preferred_element_type=jnp.float32)
    zq_ref[0] = zq
    diff = zblk - zq
    lacc_ref[0, 0] += jnp.sum(diff * diff)

    mm_ref[(i + 1) % 2] = lax.dot_general(
        ebf_ref[...], zd_ref[0].astype(jnp.bfloat16),
        (((1,), (0,)), ((), ())), preferred_element_type=jnp.float32)

    @pl.when(i == nsteps - 1)
    def _fin():
        avg = counts_ref[...] * (1.0 / ntok)
        ent = jnp.sum(avg * jnp.log(avg + _EPS))
        pp_ref[0, 0] = jnp.exp(-ent)
        loss_ref[0, 0] = lacc_ref[0, 0] * (_BETA / (ntok * cdim))


def kernel(z, embedding):
    B, C, D, H, W = z.shape
    K = embedding.shape[0]
    S = D * H * W
    N = B * S
    TB = 2048
    NB = N // TB
    SB = S // TB

    z3 = z.reshape(B, C, S)
    e2 = jnp.sum(embedding ** 2, axis=1, keepdims=True)

    out_shape = (
        jax.ShapeDtypeStruct((B, C, S), jnp.float32),
        jax.ShapeDtypeStruct((NB, 1, TB), jnp.int32),
        jax.ShapeDtypeStruct((1, 1), jnp.float32),
        jax.ShapeDtypeStruct((1, 1), jnp.float32),
    )

    def _blk(i):
        return (i // SB, 0, i % SB)

    def _blk_next(i):
        j = jnp.minimum(i + 1, NB - 1)
        return (j // SB, 0, j % SB)

    in_specs = [
        pl.BlockSpec((K, C), lambda i: (0, 0)),
        pl.BlockSpec((1, C, TB), _blk),
        pl.BlockSpec((1, C, TB), _blk_next),
        pl.BlockSpec((K, 1), lambda i: (0, 0)),
    ]
    out_specs = (
        pl.BlockSpec((1, C, TB), _blk),
        pl.BlockSpec((1, 1, TB), lambda i: (i, 0, 0)),
        pl.BlockSpec(memory_space=pltpu.SMEM),
        pl.BlockSpec(memory_space=pltpu.SMEM),
    )
    zq3, idxb, loss, pp = pl.pallas_call(
        _vq_body,
        grid=(NB,),
        in_specs=in_specs,
        out_specs=out_specs,
        out_shape=out_shape,
        scratch_shapes=[pltpu.VMEM((K, 1), jnp.float32),
                        pltpu.SMEM((1, 1), jnp.float32),
                        pltpu.VMEM((K, C), jnp.bfloat16),
                        pltpu.VMEM((2, K, TB), jnp.float32)],
    )(embedding, z3, z3, e2)

    z_q = zq3.reshape(B, C, D, H, W)
    indices = idxb.reshape(B, D, H, W)
    return (z_q, indices, loss[0, 0], pp[0, 0])


# speedup vs baseline: 1.0186x; 1.0186x over previous
import jax
import jax.numpy as jnp
from jax import lax
from jax.experimental import pallas as pl
from jax.experimental.pallas import tpu as pltpu

_BETA = 0.25
_EPS = 1e-10


def _vq_body(e_ref, ze_ref, zd_ref, e2_ref, zq_ref, idx_ref, loss_ref, pp_ref,
             counts_ref, lacc_ref, ebf_ref, mm_ref, iota_ref):
    i = pl.program_id(0)
    nsteps = pl.num_programs(0)

    e2 = e2_ref[...]
    kcodes = e2.shape[0]
    cdim, tb = ze_ref.shape[1], ze_ref.shape[2]
    ntok = nsteps * tb

    @pl.when(i == 0)
    def _init():
        counts_ref[...] = jnp.zeros_like(counts_ref)
        lacc_ref[0, 0] = 0.0
        ebf_ref[...] = e_ref[...].astype(jnp.bfloat16)
        iota_ref[...] = lax.broadcasted_iota(
            jnp.int32, iota_ref.shape, 0).astype(jnp.float32)
        mm_ref[0] = lax.dot_general(
            ebf_ref[...], ze_ref[0].astype(jnp.bfloat16),
            (((1,), (0,)), ((), ())), preferred_element_type=jnp.float32)

    zblk = ze_ref[0]
    mm = mm_ref[i % 2]
    z2k = jnp.sum(zblk * zblk, axis=0, keepdims=True)
    dist = (z2k + e2) - 2.0 * mm
    minv = jnp.min(dist, axis=0, keepdims=True)
    iota = iota_ref[...]
    idxf = jnp.min(jnp.where(dist == minv, iota, float(kcodes)),
                   axis=0, keepdims=True)
    idx_ref[0] = idxf.astype(jnp.int32)

    onehot = (iota == idxf).astype(jnp.bfloat16)
    ones = jnp.ones((tb, 1), jnp.bfloat16)
    counts_ref[...] += lax.dot_general(
        onehot, ones, (((1,), (0,)), ((), ())),
        preferred_element_type=jnp.float32)
    zq = lax.dot_general(ebf_ref[...], onehot, (((0,), (0,)), ((), ())),
                         preferred_element_type=jnp.float32)
    zq_ref[0] = zq
    diff = zblk - zq
    lacc_ref[0, 0] += jnp.sum(diff * diff)

    mm_ref[(i + 1) % 2] = lax.dot_general(
        ebf_ref[...], zd_ref[0].astype(jnp.bfloat16),
        (((1,), (0,)), ((), ())), preferred_element_type=jnp.float32)

    @pl.when(i == nsteps - 1)
    def _fin():
        avg = counts_ref[...] * (1.0 / ntok)
        ent = jnp.sum(avg * jnp.log(avg + _EPS))
        pp_ref[0, 0] = jnp.exp(-ent)
        loss_ref[0, 0] = lacc_ref[0, 0] * (_BETA / (ntok * cdim))


def kernel(z, embedding):
    B, C, D, H, W = z.shape
    K = embedding.shape[0]
    S = D * H * W
    N = B * S
    TB = 1024
    NB = N // TB
    SB = S // TB

    z3 = z.reshape(B, C, S)
    e2 = jnp.sum(embedding ** 2, axis=1, keepdims=True)

    out_shape = (
        jax.ShapeDtypeStruct((B, C, S), jnp.float32),
        jax.ShapeDtypeStruct((NB, 1, TB), jnp.int32),
        jax.ShapeDtypeStruct((1, 1), jnp.float32),
        jax.ShapeDtypeStruct((1, 1), jnp.float32),
    )

    def _blk(i):
        return (i // SB, 0, i % SB)

    def _blk_next(i):
        j = jnp.minimum(i + 1, NB - 1)
        return (j // SB, 0, j % SB)

    in_specs = [
        pl.BlockSpec((K, C), lambda i: (0, 0)),
        pl.BlockSpec((1, C, TB), _blk),
        pl.BlockSpec((1, C, TB), _blk_next),
        pl.BlockSpec((K, 1), lambda i: (0, 0)),
    ]
    out_specs = (
        pl.BlockSpec((1, C, TB), _blk),
        pl.BlockSpec((1, 1, TB), lambda i: (i, 0, 0)),
        pl.BlockSpec(memory_space=pltpu.SMEM),
        pl.BlockSpec(memory_space=pltpu.SMEM),
    )
    zq3, idxb, loss, pp = pl.pallas_call(
        _vq_body,
        grid=(NB,),
        in_specs=in_specs,
        out_specs=out_specs,
        out_shape=out_shape,
        scratch_shapes=[pltpu.VMEM((K, 1), jnp.float32),
                        pltpu.SMEM((1, 1), jnp.float32),
                        pltpu.VMEM((K, C), jnp.bfloat16),
                        pltpu.VMEM((2, K, TB), jnp.float32),
                        pltpu.VMEM((K, TB), jnp.float32)],
    )(embedding, z3, z3, e2)

    z_q = zq3.reshape(B, C, D, H, W)
    indices = idxb.reshape(B, D, H, W)
    return (z_q, indices, loss[0, 0], pp[0, 0])
